# Initial kernel scaffold; baseline (speedup 1.0000x reference)
import jax
import jax.numpy as jnp
from jax.experimental import pallas as pl


def kernel(node_ids, messages, timestamps):
    raise NotImplementedError("write your pallas kernel here")


# speedup vs baseline: 3.5569x; 3.5569x over previous
import functools

import jax
import jax.numpy as jnp
from jax import lax
from jax.experimental import pallas as pl
from jax.experimental.pallas import tpu as pltpu
from jax.experimental.pallas import tpu_sc as plsc

_N_NODES = 10000
_N_PAD = 10240
_D = 128
_LAMB_INV = 1.0 / 10.0
_L = 16
_NC = 2
_NS = 16
_BLK = 80
_PER_TILE = _N_PAD // _NS


def _sc_body(nid_hbm, ts_hbm, msg_hbm, part_hbm, tpad_hbm, hpad_hbm,
             nb96, nb80, tb, valb, wb, mb, tlb, ob, acc_out, acc_t):
    cid = lax.axis_index("c")
    sid = lax.axis_index("s")
    n_msg = nid_hbm.shape[0]
    ca = n_msg // _NS
    cb = n_msg // (_NC * _NS)
    na_blocks = ca // _BLK
    nb_blocks = cb // _BLK
    zero16 = jnp.zeros((_L,), jnp.float32)

    for g in range(_PER_TILE // _L):
        ob[pl.ds(g * _L, _L)] = zero16

    def zrow(r, carry):
        for c in range(_D // _L):
            mb[r, pl.ds(c * _L, _L)] = zero16
        return carry

    lax.fori_loop(0, _BLK, zrow, 0)
    pltpu.sync_copy(ob, acc_t.at[pl.ds(sid * _PER_TILE, _PER_TILE)])
    for k in range(_PER_TILE // _BLK):
        pltpu.sync_copy(mb, acc_out.at[pl.ds(sid * _PER_TILE + k * _BLK, _BLK), :])
    plsc.subcore_barrier()

    def phase_a(i, carry):
        start = sid * ca + i * _BLK
        pltpu.sync_copy(nid_hbm.at[pl.ds(start, _BLK)], nb96.at[pl.ds(0, _BLK)])
        pltpu.sync_copy(nid_hbm.at[pl.ds(start, _BLK)], nb80)
        pltpu.sync_copy(ts_hbm.at[pl.ds(start, _BLK)], tb)
        is_last = start + _BLK >= n_msg

        @pl.when(jnp.logical_not(is_last))
        def _():
            pltpu.sync_copy(nid_hbm.at[pl.ds(start + _BLK, _L)],
                            nb96.at[pl.ds(_BLK, _L)])

        @pl.when(is_last)
        def _():
            nb96[pl.ds(_BLK, _L)] = jnp.full((_L,), -1, jnp.int32)

        for g in range(_BLK // _L):
            cur = nb96[pl.ds(g * _L, _L)]
            nxt = nb96[pl.ds(g * _L + 1, _L)]
            tsv = tb[pl.ds(g * _L, _L)]
            valb[pl.ds(g * _L, _L)] = jnp.where(cur != nxt, tsv + 1.0, 0.0)
        pltpu.sync_copy(valb, acc_t.at[nb80], add=True)
        return carry

    lax.fori_loop(0, na_blocks, phase_a, 0)
    plsc.subcore_barrier()

    pltpu.sync_copy(acc_t, tlb)
    base_n = sid * _PER_TILE

    @pl.when(cid == 0)
    def _():
        for g in range(_PER_TILE // _L):
            v = tlb[pl.ds(base_n + g * _L, _L)]
            ob[pl.ds(g * _L, _L)] = jnp.maximum(v - 1.0, 0.0)
        pltpu.sync_copy(ob, tpad_hbm.at[pl.ds(base_n, _PER_TILE)])
        for g in range(_PER_TILE // _L):
            v = tlb[pl.ds(base_n + g * _L, _L)]
            ob[pl.ds(g * _L, _L)] = jnp.where(v > 0.0, 1.0, 0.0)
        pltpu.sync_copy(ob, hpad_hbm.at[pl.ds(base_n, _PER_TILE)])

    def phase_b(i, carry):
        base = cid * (n_msg // _NC) + sid * cb + i * _BLK
        pltpu.sync_copy(nid_hbm.at[pl.ds(base, _BLK)], nb80)
        pltpu.sync_copy(ts_hbm.at[pl.ds(base, _BLK)], tb)
        pltpu.sync_copy(msg_hbm.at[pl.ds(base, _BLK), :], mb)
        for g in range(_BLK // _L):
            idxv = nb80[pl.ds(g * _L, _L)]
            tl1 = plsc.load_gather(tlb, [idxv])
            w = jnp.exp((tb[pl.ds(g * _L, _L)] - (tl1 - 1.0)) * _LAMB_INV)
            wb[pl.ds(g * _L, _L)] = w

        def row_fn(r, inner):
            wsp = plsc.load_gather(wb, [lax.broadcast(r, (_L,))])
            for c in range(_D // _L):
                mb[r, pl.ds(c * _L, _L)] = mb[r, pl.ds(c * _L, _L)] * wsp
            return inner

        lax.fori_loop(0, _BLK, row_fn, 0)
        pltpu.sync_copy(mb, acc_out.at[nb80], add=True)
        return carry

    lax.fori_loop(0, nb_blocks, phase_b, 0)
    plsc.subcore_barrier()

    pltpu.sync_copy(acc_out.at[pl.ds(sid * _PER_TILE, _PER_TILE), :],
                    part_hbm.at[cid, pl.ds(sid * _PER_TILE, _PER_TILE), :])


@jax.jit
def _sc_call(node_ids, timestamps, messages):
    mesh = plsc.VectorSubcoreMesh(core_axis_name="c", subcore_axis_name="s",
                                  num_cores=_NC, num_subcores=_NS)
    fn = pl.kernel(
        _sc_body,
        out_type=(
            jax.ShapeDtypeStruct((_NC, _N_PAD, _D), jnp.float32),
            jax.ShapeDtypeStruct((_N_PAD,), jnp.float32),
            jax.ShapeDtypeStruct((_N_PAD,), jnp.float32),
        ),
        mesh=mesh,
        compiler_params=pltpu.CompilerParams(needs_layout_passes=False),
        scratch_types=[
            pltpu.VMEM((96,), jnp.int32),
            pltpu.VMEM((_BLK,), jnp.int32),
            pltpu.VMEM((_BLK,), jnp.float32),
            pltpu.VMEM((_BLK,), jnp.float32),
            pltpu.VMEM((_BLK,), jnp.float32),
            pltpu.VMEM((_BLK, _D), jnp.float32),
            pltpu.VMEM((_N_PAD,), jnp.float32),
            pltpu.VMEM((_PER_TILE,), jnp.float32),
            pltpu.VMEM_SHARED((_N_PAD, _D), jnp.float32),
            pltpu.VMEM_SHARED((_N_PAD,), jnp.float32),
        ],
    )
    return fn(node_ids, timestamps, messages)


def _combine_body(p_ref, o_ref):
    o_ref[...] = p_ref[0] + p_ref[1]


@jax.jit
def _combine(part):
    return pl.pallas_call(
        _combine_body,
        grid=(_N_PAD // _PER_TILE,),
        in_specs=[pl.BlockSpec((_NC, _PER_TILE, _D), lambda i: (0, i, 0))],
        out_specs=pl.BlockSpec((_PER_TILE, _D), lambda i: (i, 0)),
        out_shape=jax.ShapeDtypeStruct((_N_PAD, _D), jnp.float32),
    )(part)


def kernel(node_ids, messages, timestamps):
    part, tpad, hpad = _sc_call(node_ids.astype(jnp.int32),
                                timestamps.astype(jnp.float32),
                                messages)
    out = _combine(part)
    return (hpad[:_N_NODES] > 0.5,
            out[:_N_NODES],
            tpad[:_N_NODES])


# speedup vs baseline: 10.6500x; 2.9942x over previous
import functools

import jax
import jax.numpy as jnp
from jax import lax
from jax.experimental import pallas as pl
from jax.experimental.pallas import tpu as pltpu
from jax.experimental.pallas import tpu_sc as plsc

_N_NODES = 10000
_N_PAD = 10240
_D = 128
_LAMB_INV = 1.0 / 10.0
_L = 16
_NC = 2
_NS = 16
_BLK = 80
_SUP = 80
_PER_TILE = _N_PAD // _NS


def _dbuf_loop(nblocks, start_fn, proc_fn):
    start_fn(0, 0)

    def body(i, carry):
        @pl.when(i % 2 == 0)
        def _():
            @pl.when(i + 1 < nblocks)
            def _():
                start_fn(i + 1, 1)
            proc_fn(i, 0)

        @pl.when(i % 2 == 1)
        def _():
            @pl.when(i + 1 < nblocks)
            def _():
                start_fn(i + 1, 0)
            proc_fn(i, 1)

        return carry

    lax.fori_loop(0, nblocks, body, 0)


def _sc_body(nid_hbm, nid3d_hbm, ts_hbm, msg_hbm, part_hbm, tpad_hbm,
             hpad_hbm, na0, na1, nbs0, nbs1, tb0, tb1, mb0, mb1,
             valb, wb, tlb, ob, sem0, sem1, acc_out, acc_t):
    cid = lax.axis_index("c")
    sid = lax.axis_index("s")
    n_msg = nid_hbm.shape[0]
    ca = n_msg // _NS
    cb = n_msg // (_NC * _NS)
    na_blocks = ca // _SUP
    nb_blocks = cb // _SUP
    zero16 = jnp.zeros((_L,), jnp.float32)
    nas = (na0, na1)
    nbss = (nbs0, nbs1)
    tbs = (tb0, tb1)
    mbs = (mb0, mb1)
    sems = (sem0, sem1)

    for g in range(_PER_TILE // _L):
        ob[pl.ds(g * _L, _L)] = zero16

    def zrow(r, carry):
        for c in range(_D // _L):
            mb0[r, pl.ds(c * _L, _L)] = zero16
        return carry

    lax.fori_loop(0, _SUP, zrow, 0)
    tile0 = pl.multiple_of(sid * _PER_TILE, 8)
    pltpu.sync_copy(ob, acc_t.at[pl.ds(tile0, _PER_TILE)])
    for k in range(_PER_TILE // _SUP):
        pltpu.sync_copy(mb0.at[pl.ds(0, _SUP), :],
                        acc_out.at[pl.ds(tile0 + k * _SUP, _SUP), :])
    plsc.subcore_barrier()

    def a_start(i, buf):
        start = pl.multiple_of(sid * ca + i * _SUP, 8)
        la = pl.multiple_of(jnp.minimum(start + _SUP, n_msg - _L), 8)
        pltpu.async_copy(nid_hbm.at[pl.ds(start, _SUP)],
                         nas[buf].at[pl.ds(0, _SUP)], sems[buf])
        pltpu.async_copy(nid_hbm.at[pl.ds(la, _L)],
                         nas[buf].at[pl.ds(_SUP, _L)], sems[buf])
        pltpu.async_copy(nid3d_hbm.at[start // _SUP], nbss[buf], sems[buf])
        pltpu.async_copy(ts_hbm.at[pl.ds(start, _SUP)], tbs[buf], sems[buf])

    def a_proc(i, buf):
        start = pl.multiple_of(sid * ca + i * _SUP, 8)
        la = pl.multiple_of(jnp.minimum(start + _SUP, n_msg - _L), 8)
        pltpu.make_async_copy(nid_hbm.at[pl.ds(start, _SUP)],
                              nas[buf].at[pl.ds(0, _SUP)], sems[buf]).wait()
        pltpu.make_async_copy(nid_hbm.at[pl.ds(la, _L)],
                              nas[buf].at[pl.ds(_SUP, _L)], sems[buf]).wait()
        pltpu.make_async_copy(nid3d_hbm.at[start // _SUP],
                              nbss[buf], sems[buf]).wait()
        pltpu.make_async_copy(ts_hbm.at[pl.ds(start, _SUP)], tbs[buf],
                              sems[buf]).wait()

        @pl.when(start + _SUP >= n_msg)
        def _():
            nas[buf][pl.ds(_SUP, _L)] = jnp.full((_L,), -1, jnp.int32)

        for g in range(_SUP // _L):
            cur = nas[buf][pl.ds(g * _L, _L)]
            nxt = nas[buf][pl.ds(g * _L + 1, _L)]
            tsv = tbs[buf][pl.ds(g * _L, _L)]
            valb[g // (_BLK // _L), pl.ds((g % (_BLK // _L)) * _L, _L)] = (
                jnp.where(cur != nxt, tsv + 1.0, 0.0))
        for j in range(_SUP // _BLK):
            pltpu.sync_copy(valb.at[j], acc_t.at[nbss[buf].at[j]], add=True)

    _dbuf_loop(na_blocks, a_start, a_proc)
    plsc.subcore_barrier()

    pltpu.sync_copy(acc_t, tlb)
    base_n = pl.multiple_of(sid * _PER_TILE, 8)

    @pl.when(cid == 0)
    def _():
        for g in range(_PER_TILE // _L):
            v = tlb[pl.ds(base_n + g * _L, _L)]
            ob[pl.ds(g * _L, _L)] = jnp.maximum(v - 1.0, 0.0)
        pltpu.sync_copy(ob, tpad_hbm.at[pl.ds(base_n, _PER_TILE)])
        for g in range(_PER_TILE // _L):
            v = tlb[pl.ds(base_n + g * _L, _L)]
            ob[pl.ds(g * _L, _L)] = jnp.where(v > 0.0, 1.0, 0.0)
        pltpu.sync_copy(ob, hpad_hbm.at[pl.ds(base_n, _PER_TILE)])

    def b_start(i, buf):
        base = pl.multiple_of(cid * (n_msg // _NC) + sid * cb + i * _SUP, 8)
        pltpu.async_copy(nid3d_hbm.at[base // _SUP], nbss[buf], sems[buf])
        pltpu.async_copy(ts_hbm.at[pl.ds(base, _SUP)], tbs[buf], sems[buf])
        pltpu.async_copy(msg_hbm.at[pl.ds(base, _SUP), :], mbs[buf], sems[buf])

    def b_proc(i, buf):
        base = pl.multiple_of(cid * (n_msg // _NC) + sid * cb + i * _SUP, 8)
        pltpu.make_async_copy(nid3d_hbm.at[base // _SUP],
                              nbss[buf], sems[buf]).wait()
        pltpu.make_async_copy(ts_hbm.at[pl.ds(base, _SUP)], tbs[buf],
                              sems[buf]).wait()
        pltpu.make_async_copy(msg_hbm.at[pl.ds(base, _SUP), :], mbs[buf],
                              sems[buf]).wait()
        for g in range(_SUP // _L):
            idxv = nbss[buf][g // (_BLK // _L),
                             pl.ds((g % (_BLK // _L)) * _L, _L)]
            tl1 = plsc.load_gather(tlb, [idxv])
            w = jnp.exp((tbs[buf][pl.ds(g * _L, _L)] - (tl1 - 1.0)) * _LAMB_INV)
            wb[pl.ds(g * _L, _L)] = w

        def row_fn(r, inner):
            wsp = plsc.load_gather(wb, [lax.broadcast(r, (_L,))])
            for c in range(_D // _L):
                mbs[buf][r, pl.ds(c * _L, _L)] = (
                    mbs[buf][r, pl.ds(c * _L, _L)] * wsp)
            return inner

        lax.fori_loop(0, _SUP, row_fn, 0)
        for j in range(_SUP // _BLK):
            pltpu.sync_copy(mbs[buf].at[pl.ds(j * _BLK, _BLK), :],
                            acc_out.at[nbss[buf].at[j]], add=True)

    _dbuf_loop(nb_blocks, b_start, b_proc)
    plsc.subcore_barrier()

    out0 = pl.multiple_of(sid * _PER_TILE, 8)
    pltpu.sync_copy(acc_out.at[pl.ds(out0, _PER_TILE), :],
                    part_hbm.at[cid, pl.ds(out0, _PER_TILE), :])


@jax.jit
def _sc_call(node_ids, timestamps, messages):
    mesh = plsc.VectorSubcoreMesh(core_axis_name="c", subcore_axis_name="s",
                                  num_cores=_NC, num_subcores=_NS)
    fn = pl.kernel(
        _sc_body,
        out_type=(
            jax.ShapeDtypeStruct((_NC, _N_PAD, _D), jnp.float32),
            jax.ShapeDtypeStruct((_N_PAD,), jnp.float32),
            jax.ShapeDtypeStruct((_N_PAD,), jnp.float32),
        ),
        mesh=mesh,
        compiler_params=pltpu.CompilerParams(needs_layout_passes=False),
        scratch_types=[
            pltpu.VMEM((_SUP + _L,), jnp.int32),
            pltpu.VMEM((_SUP + _L,), jnp.int32),
            pltpu.VMEM((_SUP // _BLK, _BLK), jnp.int32),
            pltpu.VMEM((_SUP // _BLK, _BLK), jnp.int32),
            pltpu.VMEM((_SUP,), jnp.float32),
            pltpu.VMEM((_SUP,), jnp.float32),
            pltpu.VMEM((_SUP, _D), jnp.float32),
            pltpu.VMEM((_SUP, _D), jnp.float32),
            pltpu.VMEM((_SUP // _BLK, _BLK), jnp.float32),
            pltpu.VMEM((_SUP,), jnp.float32),
            pltpu.VMEM((_N_PAD,), jnp.float32),
            pltpu.VMEM((_PER_TILE,), jnp.float32),
            pltpu.SemaphoreType.DMA,
            pltpu.SemaphoreType.DMA,
            pltpu.VMEM_SHARED((_N_PAD, _D), jnp.float32),
            pltpu.VMEM_SHARED((_N_PAD,), jnp.float32),
        ],
    )
    return fn(node_ids, node_ids.reshape(-1, _SUP // _BLK, _BLK), timestamps, messages)


def _combine_body(p_ref, o_ref):
    o_ref[...] = p_ref[0] + p_ref[1]


@jax.jit
def _combine(part):
    return pl.pallas_call(
        _combine_body,
        grid=(_N_PAD // _PER_TILE,),
        in_specs=[pl.BlockSpec((_NC, _PER_TILE, _D), lambda i: (0, i, 0))],
        out_specs=pl.BlockSpec((_PER_TILE, _D), lambda i: (i, 0)),
        out_shape=jax.ShapeDtypeStruct((_N_PAD, _D), jnp.float32),
    )(part)


def kernel(node_ids, messages, timestamps):
    part, tpad, hpad = _sc_call(node_ids.astype(jnp.int32),
                                timestamps.astype(jnp.float32),
                                messages)
    out = _combine(part)
    return (hpad[:_N_NODES] > 0.5,
            out[:_N_NODES],
            tpad[:_N_NODES])


# speedup vs baseline: 12.1570x; 1.1415x over previous
import functools

import jax
import jax.numpy as jnp
from jax import lax
from jax.experimental import pallas as pl
from jax.experimental.pallas import tpu as pltpu
from jax.experimental.pallas import tpu_sc as plsc

_N_NODES = 10000
_N_PAD = 10240
_D = 128
_LAMB_INV = 1.0 / 10.0
_L = 16
_NC = 2
_NS = 16
_BLK = 80
_SUP = 80
_PER_TILE = _N_PAD // _NS


def _dbuf_loop(nblocks, start_fn, proc_fn):
    start_fn(0, 0)

    def body(i, carry):
        @pl.when(i % 2 == 0)
        def _():
            @pl.when(i + 1 < nblocks)
            def _():
                start_fn(i + 1, 1)
            proc_fn(i, 0)

        @pl.when(i % 2 == 1)
        def _():
            @pl.when(i + 1 < nblocks)
            def _():
                start_fn(i + 1, 0)
            proc_fn(i, 1)

        return carry

    lax.fori_loop(0, nblocks, body, 0)


def _sc_body(nid_hbm, nid3d_hbm, ts_hbm, msg_hbm, part_hbm, tpad_hbm,
             hpad_hbm, na0, na1, nbs0, nbs1, tb0, tb1, mb0, mb1,
             valb0, valb1, wb, tlb, ob, sem0, sem1, ssem0, ssem1,
             acc_out, acc_t):
    cid = lax.axis_index("c")
    sid = lax.axis_index("s")
    n_msg = nid_hbm.shape[0]
    ca = n_msg // _NS
    cb = n_msg // (_NC * _NS)
    na_blocks = ca // _SUP
    nb_blocks = cb // _SUP
    zero16 = jnp.zeros((_L,), jnp.float32)
    nas = (na0, na1)
    nbss = (nbs0, nbs1)
    tbs = (tb0, tb1)
    mbs = (mb0, mb1)
    sems = (sem0, sem1)
    valbs = (valb0, valb1)
    ssems = (ssem0, ssem1)

    for g in range(_PER_TILE // _L):
        ob[pl.ds(g * _L, _L)] = zero16

    def zrow(r, carry):
        for c in range(_D // _L):
            mb0[r, pl.ds(c * _L, _L)] = zero16
        return carry

    lax.fori_loop(0, _SUP, zrow, 0)
    tile0 = pl.multiple_of(sid * _PER_TILE, 8)
    pltpu.sync_copy(ob, acc_t.at[pl.ds(tile0, _PER_TILE)])
    for k in range(_PER_TILE // _SUP):
        pltpu.sync_copy(mb0.at[pl.ds(0, _SUP), :],
                        acc_out.at[pl.ds(tile0 + k * _SUP, _SUP), :])
    plsc.subcore_barrier()

    def a_start(i, buf):
        if not (isinstance(i, int) and i < 2):
            @pl.when(i >= 2)
            def _():
                pltpu.make_async_copy(valbs[buf].at[0],
                                      acc_t.at[nbss[buf].at[0]],
                                      ssems[buf]).wait()
        start = pl.multiple_of(sid * ca + i * _SUP, 8)
        la = pl.multiple_of(jnp.minimum(start + _SUP, n_msg - _L), 8)
        pltpu.async_copy(nid_hbm.at[pl.ds(start, _SUP)],
                         nas[buf].at[pl.ds(0, _SUP)], sems[buf])
        pltpu.async_copy(nid_hbm.at[pl.ds(la, _L)],
                         nas[buf].at[pl.ds(_SUP, _L)], sems[buf])
        pltpu.async_copy(nid3d_hbm.at[start // _SUP], nbss[buf], sems[buf])
        pltpu.async_copy(ts_hbm.at[pl.ds(start, _SUP)], tbs[buf], sems[buf])

    def a_proc(i, buf):
        start = pl.multiple_of(sid * ca + i * _SUP, 8)
        la = pl.multiple_of(jnp.minimum(start + _SUP, n_msg - _L), 8)
        pltpu.make_async_copy(nid_hbm.at[pl.ds(start, _SUP)],
                              nas[buf].at[pl.ds(0, _SUP)], sems[buf]).wait()
        pltpu.make_async_copy(nid_hbm.at[pl.ds(la, _L)],
                              nas[buf].at[pl.ds(_SUP, _L)], sems[buf]).wait()
        pltpu.make_async_copy(nid3d_hbm.at[start // _SUP],
                              nbss[buf], sems[buf]).wait()
        pltpu.make_async_copy(ts_hbm.at[pl.ds(start, _SUP)], tbs[buf],
                              sems[buf]).wait()

        @pl.when(start + _SUP >= n_msg)
        def _():
            nas[buf][pl.ds(_SUP, _L)] = jnp.full((_L,), -1, jnp.int32)

        for g in range(_SUP // _L):
            cur = nas[buf][pl.ds(g * _L, _L)]
            nxt = nas[buf][pl.ds(g * _L + 1, _L)]
            tsv = tbs[buf][pl.ds(g * _L, _L)]
            valbs[buf][g // (_BLK // _L),
                       pl.ds((g % (_BLK // _L)) * _L, _L)] = (
                jnp.where(cur != nxt, tsv + 1.0, 0.0))
        pltpu.async_copy(valbs[buf].at[0], acc_t.at[nbss[buf].at[0]],
                         ssems[buf], add=True)

    _dbuf_loop(na_blocks, a_start, a_proc)
    for buf in range(2):
        pltpu.make_async_copy(valbs[buf].at[0], acc_t.at[nbss[buf].at[0]],
                              ssems[buf]).wait()
    plsc.subcore_barrier()

    pltpu.sync_copy(acc_t, tlb)
    base_n = pl.multiple_of(sid * _PER_TILE, 8)

    @pl.when(cid == 0)
    def _():
        for g in range(_PER_TILE // _L):
            v = tlb[pl.ds(base_n + g * _L, _L)]
            ob[pl.ds(g * _L, _L)] = jnp.maximum(v - 1.0, 0.0)
        pltpu.sync_copy(ob, tpad_hbm.at[pl.ds(base_n, _PER_TILE)])
        for g in range(_PER_TILE // _L):
            v = tlb[pl.ds(base_n + g * _L, _L)]
            ob[pl.ds(g * _L, _L)] = jnp.where(v > 0.0, 1.0, 0.0)
        pltpu.sync_copy(ob, hpad_hbm.at[pl.ds(base_n, _PER_TILE)])

    def b_start(i, buf):
        if not (isinstance(i, int) and i < 2):
            @pl.when(i >= 2)
            def _():
                pltpu.make_async_copy(mbs[buf], acc_out.at[nbss[buf].at[0]],
                                      ssems[buf]).wait()
        base = pl.multiple_of(cid * (n_msg // _NC) + sid * cb + i * _SUP, 8)
        pltpu.async_copy(nid3d_hbm.at[base // _SUP], nbss[buf], sems[buf])
        pltpu.async_copy(ts_hbm.at[pl.ds(base, _SUP)], tbs[buf], sems[buf])
        pltpu.async_copy(msg_hbm.at[pl.ds(base, _SUP), :], mbs[buf], sems[buf])

    def b_proc(i, buf):
        base = pl.multiple_of(cid * (n_msg // _NC) + sid * cb + i * _SUP, 8)
        pltpu.make_async_copy(nid3d_hbm.at[base // _SUP],
                              nbss[buf], sems[buf]).wait()
        pltpu.make_async_copy(ts_hbm.at[pl.ds(base, _SUP)], tbs[buf],
                              sems[buf]).wait()
        pltpu.make_async_copy(msg_hbm.at[pl.ds(base, _SUP), :], mbs[buf],
                              sems[buf]).wait()
        for g in range(_SUP // _L):
            idxv = nbss[buf][g // (_BLK // _L),
                             pl.ds((g % (_BLK // _L)) * _L, _L)]
            tl1 = plsc.load_gather(tlb, [idxv])
            w = jnp.exp((tbs[buf][pl.ds(g * _L, _L)] - (tl1 - 1.0)) * _LAMB_INV)
            wb[pl.ds(g * _L, _L)] = w

        @plsc.parallel_loop(0, _SUP, unroll=2)
        def row_fn(r):
            wsp = plsc.load_gather(wb, [lax.broadcast(r, (_L,))])
            for c in range(_D // _L):
                mbs[buf][r, pl.ds(c * _L, _L)] = (
                    mbs[buf][r, pl.ds(c * _L, _L)] * wsp)

        pltpu.async_copy(mbs[buf], acc_out.at[nbss[buf].at[0]],
                         ssems[buf], add=True)

    _dbuf_loop(nb_blocks, b_start, b_proc)
    for buf in range(2):
        pltpu.make_async_copy(mbs[buf], acc_out.at[nbss[buf].at[0]],
                              ssems[buf]).wait()
    plsc.subcore_barrier()

    out0 = pl.multiple_of(sid * _PER_TILE, 8)
    pltpu.sync_copy(acc_out.at[pl.ds(out0, _PER_TILE), :],
                    part_hbm.at[cid, pl.ds(out0, _PER_TILE), :])


@jax.jit
def _sc_call(node_ids, timestamps, messages):
    mesh = plsc.VectorSubcoreMesh(core_axis_name="c", subcore_axis_name="s",
                                  num_cores=_NC, num_subcores=_NS)
    fn = pl.kernel(
        _sc_body,
        out_type=(
            jax.ShapeDtypeStruct((_NC, _N_PAD, _D), jnp.float32),
            jax.ShapeDtypeStruct((_N_PAD,), jnp.float32),
            jax.ShapeDtypeStruct((_N_PAD,), jnp.float32),
        ),
        mesh=mesh,
        compiler_params=pltpu.CompilerParams(needs_layout_passes=False),
        scratch_types=[
            pltpu.VMEM((_SUP + _L,), jnp.int32),
            pltpu.VMEM((_SUP + _L,), jnp.int32),
            pltpu.VMEM((_SUP // _BLK, _BLK), jnp.int32),
            pltpu.VMEM((_SUP // _BLK, _BLK), jnp.int32),
            pltpu.VMEM((_SUP,), jnp.float32),
            pltpu.VMEM((_SUP,), jnp.float32),
            pltpu.VMEM((_SUP, _D), jnp.float32),
            pltpu.VMEM((_SUP, _D), jnp.float32),
            pltpu.VMEM((_SUP // _BLK, _BLK), jnp.float32),
            pltpu.VMEM((_SUP // _BLK, _BLK), jnp.float32),
            pltpu.VMEM((_SUP,), jnp.float32),
            pltpu.VMEM((_N_PAD,), jnp.float32),
            pltpu.VMEM((_PER_TILE,), jnp.float32),
            pltpu.SemaphoreType.DMA,
            pltpu.SemaphoreType.DMA,
            pltpu.SemaphoreType.DMA,
            pltpu.SemaphoreType.DMA,
            pltpu.VMEM_SHARED((_N_PAD, _D), jnp.float32),
            pltpu.VMEM_SHARED((_N_PAD,), jnp.float32),
        ],
    )
    return fn(node_ids, node_ids.reshape(-1, _SUP // _BLK, _BLK), timestamps, messages)


def _combine_body(p_ref, o_ref):
    o_ref[...] = p_ref[0] + p_ref[1]


@jax.jit
def _combine(part):
    return pl.pallas_call(
        _combine_body,
        grid=(_N_PAD // _PER_TILE,),
        in_specs=[pl.BlockSpec((_NC, _PER_TILE, _D), lambda i: (0, i, 0))],
        out_specs=pl.BlockSpec((_PER_TILE, _D), lambda i: (i, 0)),
        out_shape=jax.ShapeDtypeStruct((_N_PAD, _D), jnp.float32),
    )(part)


def kernel(node_ids, messages, timestamps):
    part, tpad, hpad = _sc_call(node_ids.astype(jnp.int32),
                                timestamps.astype(jnp.float32),
                                messages)
    out = _combine(part)
    return (hpad[:_N_NODES] > 0.5,
            out[:_N_NODES],
            tpad[:_N_NODES])


# speedup vs baseline: 14.9871x; 1.2328x over previous
import functools

import jax
import jax.numpy as jnp
from jax import lax
from jax.experimental import pallas as pl
from jax.experimental.pallas import tpu as pltpu
from jax.experimental.pallas import tpu_sc as plsc

_N_NODES = 10000
_N_PAD = 10240
_D = 128
_LAMB_INV = 1.0 / 10.0
_L = 16
_NC = 2
_NS = 16
_BLK = 80
_SUP = 80
_SUPA = 400
_PER_TILE = _N_PAD // _NS


def _dbuf_loop(nblocks, start_fn, proc_fn):
    start_fn(0, 0)

    def body(i, carry):
        @pl.when(i % 2 == 0)
        def _():
            @pl.when(i + 1 < nblocks)
            def _():
                start_fn(i + 1, 1)
            proc_fn(i, 0)

        @pl.when(i % 2 == 1)
        def _():
            @pl.when(i + 1 < nblocks)
            def _():
                start_fn(i + 1, 0)
            proc_fn(i, 1)

        return carry

    lax.fori_loop(0, nblocks, body, 0)


def _sc_body(nid_hbm, nid3d_hbm, nidA_hbm, ts_hbm, msg_hbm, part_hbm, tpad_hbm,
             hpad_hbm, na0, na1, nbs0, nbs1, tb0, tb1, mb0, mb1,
             nbB0, nbB1, valb0, valb1, wb, tlb, ob, sem0, sem1, ssem0, ssem1,
             acc_out, acc_t):
    cid = lax.axis_index("c")
    sid = lax.axis_index("s")
    n_msg = nid_hbm.shape[0]
    ca = n_msg // _NS
    cb = n_msg // (_NC * _NS)
    na_blocks = ca // _SUPA
    nb_blocks = cb // _SUP
    zero16 = jnp.zeros((_L,), jnp.float32)
    nas = (na0, na1)
    nbss = (nbs0, nbs1)
    tbs = (tb0, tb1)
    mbs = (mb0, mb1)
    sems = (sem0, sem1)
    valbs = (valb0, valb1)
    nbBs = (nbB0, nbB1)
    ssems = (ssem0, ssem1)

    for g in range(_PER_TILE // _L):
        ob[pl.ds(g * _L, _L)] = zero16

    def zrow(r, carry):
        for c in range(_D // _L):
            mb0[r, pl.ds(c * _L, _L)] = zero16
        return carry

    lax.fori_loop(0, _SUP, zrow, 0)
    tile0 = pl.multiple_of(sid * _PER_TILE, 8)
    pltpu.sync_copy(ob, acc_t.at[pl.ds(tile0, _PER_TILE)])
    for k in range(_PER_TILE // _SUP):
        pltpu.sync_copy(mb0.at[pl.ds(0, _SUP), :],
                        acc_out.at[pl.ds(tile0 + k * _SUP, _SUP), :])
    plsc.subcore_barrier()

    def a_start(i, buf):
        if not (isinstance(i, int) and i < 2):
            @pl.when(i >= 2)
            def _():
                for j in range(_SUPA // _BLK):
                    pltpu.make_async_copy(valbs[buf].at[j],
                                          acc_t.at[nbss[buf].at[j]],
                                          ssems[buf]).wait()
        start = pl.multiple_of(sid * ca + i * _SUPA, 8)
        la = pl.multiple_of(jnp.minimum(start + _SUPA, n_msg - _L), 8)
        pltpu.async_copy(nid_hbm.at[pl.ds(start, _SUPA)],
                         nas[buf].at[pl.ds(0, _SUPA)], sems[buf])
        pltpu.async_copy(nid_hbm.at[pl.ds(la, _L)],
                         nas[buf].at[pl.ds(_SUPA, _L)], sems[buf])
        pltpu.async_copy(nidA_hbm.at[start // _SUPA], nbss[buf], sems[buf])
        pltpu.async_copy(ts_hbm.at[pl.ds(start, _SUPA)], tbs[buf], sems[buf])

    def a_proc(i, buf):
        start = pl.multiple_of(sid * ca + i * _SUPA, 8)
        la = pl.multiple_of(jnp.minimum(start + _SUPA, n_msg - _L), 8)
        pltpu.make_async_copy(nid_hbm.at[pl.ds(start, _SUPA)],
                              nas[buf].at[pl.ds(0, _SUPA)], sems[buf]).wait()
        pltpu.make_async_copy(nid_hbm.at[pl.ds(la, _L)],
                              nas[buf].at[pl.ds(_SUPA, _L)], sems[buf]).wait()
        pltpu.make_async_copy(nidA_hbm.at[start // _SUPA],
                              nbss[buf], sems[buf]).wait()
        pltpu.make_async_copy(ts_hbm.at[pl.ds(start, _SUPA)], tbs[buf],
                              sems[buf]).wait()

        @pl.when(start + _SUPA >= n_msg)
        def _():
            nas[buf][pl.ds(_SUPA, _L)] = jnp.full((_L,), -1, jnp.int32)

        for g in range(_SUPA // _L):
            cur = nas[buf][pl.ds(g * _L, _L)]
            nxt = nas[buf][pl.ds(g * _L + 1, _L)]
            tsv = tbs[buf][pl.ds(g * _L, _L)]
            valbs[buf][g // (_BLK // _L),
                       pl.ds((g % (_BLK // _L)) * _L, _L)] = (
                jnp.where(cur != nxt, tsv + 1.0, 0.0))
        for j in range(_SUPA // _BLK):
            pltpu.async_copy(valbs[buf].at[j], acc_t.at[nbss[buf].at[j]],
                             ssems[buf], add=True)

    _dbuf_loop(na_blocks, a_start, a_proc)
    for buf in range(2):
        for j in range(_SUPA // _BLK):
            pltpu.make_async_copy(valbs[buf].at[j], acc_t.at[nbss[buf].at[j]],
                                  ssems[buf]).wait()
    plsc.subcore_barrier()

    pltpu.sync_copy(acc_t, tlb)
    base_n = pl.multiple_of(sid * _PER_TILE, 8)

    @pl.when(cid == 0)
    def _():
        for g in range(_PER_TILE // _L):
            v = tlb[pl.ds(base_n + g * _L, _L)]
            ob[pl.ds(g * _L, _L)] = jnp.maximum(v - 1.0, 0.0)
        pltpu.sync_copy(ob, tpad_hbm.at[pl.ds(base_n, _PER_TILE)])
        for g in range(_PER_TILE // _L):
            v = tlb[pl.ds(base_n + g * _L, _L)]
            ob[pl.ds(g * _L, _L)] = jnp.where(v > 0.0, 1.0, 0.0)
        pltpu.sync_copy(ob, hpad_hbm.at[pl.ds(base_n, _PER_TILE)])

    def b_start(i, buf):
        if not (isinstance(i, int) and i < 2):
            @pl.when(i >= 2)
            def _():
                pltpu.make_async_copy(mbs[buf], acc_out.at[nbBs[buf].at[0]],
                                      ssems[buf]).wait()
        base = pl.multiple_of(cid * (n_msg // _NC) + sid * cb + i * _SUP, 8)
        pltpu.async_copy(nid3d_hbm.at[base // _SUP], nbBs[buf], sems[buf])
        pltpu.async_copy(ts_hbm.at[pl.ds(base, _SUP)],
                         tbs[buf].at[pl.ds(0, _SUP)], sems[buf])
        pltpu.async_copy(msg_hbm.at[pl.ds(base, _SUP), :], mbs[buf], sems[buf])

    def b_proc(i, buf):
        base = pl.multiple_of(cid * (n_msg // _NC) + sid * cb + i * _SUP, 8)
        pltpu.make_async_copy(nid3d_hbm.at[base // _SUP],
                              nbBs[buf], sems[buf]).wait()
        pltpu.make_async_copy(ts_hbm.at[pl.ds(base, _SUP)],
                              tbs[buf].at[pl.ds(0, _SUP)], sems[buf]).wait()
        pltpu.make_async_copy(msg_hbm.at[pl.ds(base, _SUP), :], mbs[buf],
                              sems[buf]).wait()
        for g in range(_SUP // _L):
            idxv = nbBs[buf][0, pl.ds(g * _L, _L)]
            tl1 = plsc.load_gather(tlb, [idxv])
            w = jnp.exp((tbs[buf][pl.ds(g * _L, _L)] - (tl1 - 1.0)) * _LAMB_INV)
            wb[pl.ds(g * _L, _L)] = w

        @plsc.parallel_loop(0, _SUP, unroll=4)
        def row_fn(r):
            wsp = plsc.load_gather(wb, [lax.broadcast(r, (_L,))])
            for c in range(_D // _L):
                mbs[buf][r, pl.ds(c * _L, _L)] = (
                    mbs[buf][r, pl.ds(c * _L, _L)] * wsp)

        pltpu.async_copy(mbs[buf], acc_out.at[nbBs[buf].at[0]],
                         ssems[buf], add=True)

    _dbuf_loop(nb_blocks, b_start, b_proc)
    for buf in range(2):
        pltpu.make_async_copy(mbs[buf], acc_out.at[nbBs[buf].at[0]],
                              ssems[buf]).wait()
    plsc.subcore_barrier()

    out0 = pl.multiple_of(sid * _PER_TILE, 8)
    pltpu.sync_copy(acc_out.at[pl.ds(out0, _PER_TILE), :],
                    part_hbm.at[cid, pl.ds(out0, _PER_TILE), :])


@jax.jit
def _sc_call(node_ids, timestamps, messages):
    mesh = plsc.VectorSubcoreMesh(core_axis_name="c", subcore_axis_name="s",
                                  num_cores=_NC, num_subcores=_NS)
    fn = pl.kernel(
        _sc_body,
        out_type=(
            jax.ShapeDtypeStruct((_NC, _N_PAD, _D), jnp.float32),
            jax.ShapeDtypeStruct((_N_PAD,), jnp.float32),
            jax.ShapeDtypeStruct((_N_PAD,), jnp.float32),
        ),
        mesh=mesh,
        compiler_params=pltpu.CompilerParams(needs_layout_passes=False),
        scratch_types=[
            pltpu.VMEM((_SUPA + _L,), jnp.int32),
            pltpu.VMEM((_SUPA + _L,), jnp.int32),
            pltpu.VMEM((_SUPA // _BLK, _BLK), jnp.int32),
            pltpu.VMEM((_SUPA // _BLK, _BLK), jnp.int32),
            pltpu.VMEM((_SUPA,), jnp.float32),
            pltpu.VMEM((_SUPA,), jnp.float32),
            pltpu.VMEM((_SUP, _D), jnp.float32),
            pltpu.VMEM((_SUP, _D), jnp.float32),
            pltpu.VMEM((1, _BLK), jnp.int32),
            pltpu.VMEM((1, _BLK), jnp.int32),
            pltpu.VMEM((_SUPA // _BLK, _BLK), jnp.float32),
            pltpu.VMEM((_SUPA // _BLK, _BLK), jnp.float32),
            pltpu.VMEM((_SUP,), jnp.float32),
            pltpu.VMEM((_N_PAD,), jnp.float32),
            pltpu.VMEM((_PER_TILE,), jnp.float32),
            pltpu.SemaphoreType.DMA,
            pltpu.SemaphoreType.DMA,
            pltpu.SemaphoreType.DMA,
            pltpu.SemaphoreType.DMA,
            pltpu.VMEM_SHARED((_N_PAD, _D), jnp.float32),
            pltpu.VMEM_SHARED((_N_PAD,), jnp.float32),
        ],
    )
    return fn(node_ids, node_ids.reshape(-1, 1, _BLK),
              node_ids.reshape(-1, _SUPA // _BLK, _BLK),
              timestamps, messages)


def _combine_body(p_ref, o_ref):
    o_ref[...] = p_ref[0] + p_ref[1]


@jax.jit
def _combine(part):
    return pl.pallas_call(
        _combine_body,
        grid=(_N_PAD // _PER_TILE,),
        in_specs=[pl.BlockSpec((_NC, _PER_TILE, _D), lambda i: (0, i, 0))],
        out_specs=pl.BlockSpec((_PER_TILE, _D), lambda i: (i, 0)),
        out_shape=jax.ShapeDtypeStruct((_N_PAD, _D), jnp.float32),
    )(part)


def kernel(node_ids, messages, timestamps):
    part, tpad, hpad = _sc_call(node_ids.astype(jnp.int32),
                                timestamps.astype(jnp.float32),
                                messages)
    out = _combine(part)
    return (hpad[:_N_NODES] > 0.5,
            out[:_N_NODES],
            tpad[:_N_NODES])
